# Initial kernel scaffold; baseline (speedup 1.0000x reference)
import jax
import jax.numpy as jnp
from jax.experimental import pallas as pl


def kernel(x, Wr, W1, W2):
    raise NotImplementedError("write your pallas kernel here")


# speedup vs baseline: 1.2698x; 1.2698x over previous
import functools

import jax
import jax.numpy as jnp
from jax.experimental import pallas as pl
from jax.experimental.pallas import tpu as pltpu

L = 2048
D_MODEL = 1024
D_FF = 4096
N_EXPERTS = 8
TOP_K = 2

LANES = 128


def _router_kernel(x_ref, wrt_ref, wcomb_ref, aux_ref):
    x = x_ref[...]
    wrt = wrt_ref[...]
    logits = jnp.dot(x, wrt, preferred_element_type=jnp.float32)
    lane = jax.lax.broadcasted_iota(jnp.int32, logits.shape, 1)
    valid = lane < N_EXPERTS
    logits = jnp.where(valid, logits, -1e30)
    m = jnp.max(logits, axis=1, keepdims=True)
    p = jnp.exp(logits - m)
    p = jnp.where(valid, p, 0.0)
    probs = p / jnp.sum(p, axis=1, keepdims=True)
    t1v = jnp.max(probs, axis=1, keepdims=True)
    cand1 = jnp.where((probs == t1v) & valid, lane, LANES)
    t1i = jnp.min(cand1, axis=1, keepdims=True)
    probs_m = jnp.where(lane == t1i, -1.0, probs)
    t2v = jnp.max(probs_m, axis=1, keepdims=True)
    cand2 = jnp.where((probs_m == t2v) & valid, lane, LANES)
    t2i = jnp.min(cand2, axis=1, keepdims=True)
    denom = t1v + t2v + 1e-9
    w1 = t1v / denom
    w2 = t2v / denom
    wcomb = (jnp.where(lane == t1i, w1, 0.0)
             + jnp.where(lane == t2i, w2, 0.0))
    wcomb_ref[...] = wcomb
    tpe = jnp.sum(jnp.where(lane == t1i, 1.0, 0.0), axis=0) / L
    rp = jnp.sum(probs, axis=0) / L
    aux = N_EXPERTS * jnp.sum(tpe * rp)
    aux_ref[...] = jnp.zeros_like(aux_ref) + aux


def _ffn_kernel(x_ref, w1_ref, w2_ref, wc_ref, out_ref):
    e = pl.program_id(1)
    f = pl.program_id(2)

    @pl.when((e == 0) & (f == 0))
    def _():
        out_ref[...] = jnp.zeros_like(out_ref)

    x = x_ref[...]
    h = jnp.dot(x, w1_ref[0], preferred_element_type=jnp.float32)
    h = h * jax.nn.sigmoid(h)
    y = jnp.dot(h, w2_ref[0], preferred_element_type=jnp.float32)
    lane = jax.lax.broadcasted_iota(jnp.int32, wc_ref.shape, 1)
    wcol = jnp.sum(jnp.where(lane == e, wc_ref[...], 0.0), axis=1,
                   keepdims=True)
    out_ref[...] += y * wcol


def kernel(x, Wr, W1, W2):
    Bb, Ll, D = x.shape
    flat = x.reshape(Bb * Ll, D)

    wrt = jnp.zeros((D, LANES), dtype=jnp.float32).at[:, :N_EXPERTS].set(Wr.T)

    wcomb, aux = pl.pallas_call(
        _router_kernel,
        out_shape=(
            jax.ShapeDtypeStruct((L, LANES), jnp.float32),
            jax.ShapeDtypeStruct((8, 128), jnp.float32),
        ),
    )(flat, wrt)
    aux_loss = aux[0, 0]
    wc8 = wcomb[:, :N_EXPERTS]

    TB = 1024
    FB = 512
    grid = (L // TB, N_EXPERTS, D_FF // FB)
    out = pl.pallas_call(
        _ffn_kernel,
        grid=grid,
        in_specs=[
            pl.BlockSpec((TB, D), lambda t, e, f: (t, 0)),
            pl.BlockSpec((1, D, FB), lambda t, e, f: (e, 0, f)),
            pl.BlockSpec((1, FB, D), lambda t, e, f: (e, f, 0)),
            pl.BlockSpec((TB, N_EXPERTS), lambda t, e, f: (t, 0)),
        ],
        out_specs=pl.BlockSpec((TB, D), lambda t, e, f: (t, 0)),
        out_shape=jax.ShapeDtypeStruct((L, D), jnp.float32),
        compiler_params=pltpu.CompilerParams(
            dimension_semantics=("parallel", "arbitrary", "arbitrary"),
        ),
    )(flat, W1, W2, wc8)

    return out.reshape(Bb, Ll, D), aux_loss
